# baseline (speedup 1.0000x reference)
import functools

import jax
import jax.numpy as jnp
from jax import lax
from jax.experimental import pallas as pl
from jax.experimental.pallas import tpu as pltpu
from jax.experimental.pallas import tpu_sc as plsc

_NC = 2
_NS = 16
_NW = _NC * _NS
_NBUF = 5
_CH = 40


def _make_gather(vocab: int, d: int, b_total: int):
  nbuf, ch = _NBUF, _CH
  assert b_total % (8 * _NW) == 0
  b_per_w = b_total // _NW
  assert b_per_w % ch == 0
  nchunk = b_per_w // ch
  assert nchunk % nbuf == 0 and nchunk >= 2 * nbuf

  mesh = plsc.VectorSubcoreMesh(core_axis_name="c", subcore_axis_name="s")

  @functools.partial(
      pl.kernel,
      mesh=mesh,
      out_type=jax.ShapeDtypeStruct((b_total, d), jnp.float32),
      scratch_types=(
          [pltpu.VMEM((b_per_w,), jnp.int32)]
          + [pltpu.VMEM((ch, d), jnp.float32) for _ in range(nbuf)]
          + [pltpu.SemaphoreType.DMA for _ in range(2 * nbuf)]
      ),
  )
  def emb(table_hbm, idx_hbm, out_hbm, idx_v, *rest):
    rows = rest[:nbuf]
    gsem = rest[nbuf:2 * nbuf]
    ssem = rest[2 * nbuf:]
    wid = lax.axis_index("s") * _NC + lax.axis_index("c")
    base = wid * b_per_w
    pltpu.sync_copy(idx_hbm.at[pl.ds(base, b_per_w)], idx_v)

    def start_g(c, b):
      pltpu.async_copy(
          table_hbm.at[idx_v.at[pl.ds(c * ch, ch)]], rows[b], gsem[b])

    def wait_g(b):
      pltpu.make_async_copy(
          table_hbm.at[idx_v.at[pl.ds(0, ch)]], rows[b], gsem[b]).wait()

    def start_s(c, b):
      pltpu.async_copy(rows[b], out_hbm.at[pl.ds(base + c * ch, ch)], ssem[b])

    def wait_s(b):
      pltpu.make_async_copy(
          rows[b], out_hbm.at[pl.ds(base, ch)], ssem[b]).wait()

    def step(i, b, issue_wait, issue_gather):
      wait_g(b)
      start_s(i, b)
      if issue_gather:
        if issue_wait:
          wait_s((b - 1) % nbuf)
        start_g(i + nbuf - 1, (b - 1) % nbuf)

    for i in range(nbuf - 1):
      start_g(i, i)
    step(0, 0, issue_wait=False, issue_gather=True)

    n_steady = nchunk - nbuf
    n_loop = (n_steady // nbuf) * nbuf
    @pl.loop(0, n_loop // nbuf)
    def _(c):
      i0 = 1 + c * nbuf
      for k in range(nbuf):
        step(i0 + k, (1 + k) % nbuf, issue_wait=True, issue_gather=True)
    for i in range(1 + n_loop, nchunk - nbuf + 1):
      step(i, i % nbuf, issue_wait=True, issue_gather=True)

    for i in range(nchunk - nbuf + 1, nchunk):
      step(i, i % nbuf, issue_wait=False, issue_gather=False)
    for b in range(nbuf):
      wait_s(b)

  return emb


def kernel(actions_tokens, embedding_table):
  b, s = actions_tokens.shape
  vocab, d = embedding_table.shape
  idx = actions_tokens.reshape(-1).astype(jnp.int32)
  out = _make_gather(vocab, d, b * s)(embedding_table, idx)
  return out.reshape(b, s, d)
